# Initial kernel scaffold; baseline (speedup 1.0000x reference)
import jax
import jax.numpy as jnp
from jax.experimental import pallas as pl


def kernel(x, embed_in_weight, embed_out_weight):
    raise NotImplementedError("write your pallas kernel here")


# speedup vs baseline: 2.0578x; 2.0578x over previous
import functools

import jax
import jax.numpy as jnp
from jax import lax
from jax.experimental import pallas as pl
from jax.experimental.pallas import tpu as pltpu
from jax.experimental.pallas import tpu_sc as plsc

VOCAB = 1000000
N_EMBD = 1024
R = 128

NUM_CORES = 2
NUM_SUBCORES = 16
NW = NUM_CORES * NUM_SUBCORES

CHUNK = 80


def _make_gather(B):
  assert B % (8 * NW) == 0
  b_per_w = B // NW
  assert b_per_w % CHUNK == 0
  n_chunks = b_per_w // CHUNK
  mesh = plsc.VectorSubcoreMesh(core_axis_name="c", subcore_axis_name="s")

  @functools.partial(
      pl.kernel,
      out_type=jax.ShapeDtypeStruct((B, R), jnp.float32),
      mesh=mesh,
      scratch_types=[
          pltpu.VMEM((b_per_w,), jnp.int32),
          pltpu.VMEM((CHUNK, R), jnp.float32),
          pltpu.SemaphoreType.DMA,
      ],
  )
  def gather(table_hbm, idx_hbm, out_hbm, idx_v, rows_v, gsem):
    wid = lax.axis_index("s") * NUM_CORES + lax.axis_index("c")
    base = wid * b_per_w
    pltpu.sync_copy(idx_hbm.at[pl.ds(base, b_per_w)], idx_v)
    for c in range(n_chunks):
      pltpu.async_copy(table_hbm.at[idx_v.at[pl.ds(c * CHUNK, CHUNK)]],
                       rows_v, gsem).wait()
      pltpu.sync_copy(rows_v, out_hbm.at[pl.ds(base + c * CHUNK, CHUNK)])

  return gather


def _proj_body(h_ref, w_ref, o_ref):
  o_ref[...] = lax.dot_general(
      h_ref[...], w_ref[...],
      dimension_numbers=(((1,), (1,)), ((), ())),
      preferred_element_type=jnp.float32)


def _project(h, w, block_rows=512):
  B = h.shape[0]
  assert B % block_rows == 0
  return pl.pallas_call(
      _proj_body,
      grid=(B // block_rows,),
      in_specs=[
          pl.BlockSpec((block_rows, R), lambda i: (i, 0)),
          pl.BlockSpec((N_EMBD, R), lambda i: (0, 0)),
      ],
      out_specs=pl.BlockSpec((block_rows, N_EMBD), lambda i: (i, 0)),
      out_shape=jax.ShapeDtypeStruct((B, N_EMBD), jnp.float32),
  )(h, w)


def kernel(x, embed_in_weight, embed_out_weight):
  Bo, L = x.shape
  B = Bo * L
  xf = x.reshape(B).astype(jnp.int32)
  h = _make_gather(B)(embed_in_weight, xf)
  y = _project(h, embed_out_weight)
  return y.reshape(Bo, L, N_EMBD)


# speedup vs baseline: 2.0595x; 1.0008x over previous
import functools

import jax
import jax.numpy as jnp
from jax import lax
from jax.experimental import pallas as pl
from jax.experimental.pallas import tpu as pltpu
from jax.experimental.pallas import tpu_sc as plsc

VOCAB = 1000000
N_EMBD = 1024
R = 128

NUM_CORES = 2
NUM_SUBCORES = 16
NW = NUM_CORES * NUM_SUBCORES

CHUNK = 80


def _make_gather(B):
  assert B % (8 * NW) == 0
  b_per_w = B // NW
  assert b_per_w % CHUNK == 0
  n_chunks = b_per_w // CHUNK
  mesh = plsc.VectorSubcoreMesh(core_axis_name="c", subcore_axis_name="s")

  @functools.partial(
      pl.kernel,
      out_type=jax.ShapeDtypeStruct((B, R), jnp.float32),
      mesh=mesh,
      compiler_params=pltpu.CompilerParams(use_tc_tiling_on_sc=True),
      scratch_types=[
          pltpu.VMEM((b_per_w,), jnp.int32),
          pltpu.VMEM((CHUNK, R), jnp.float32),
          pltpu.SemaphoreType.DMA,
      ],
  )
  def gather(table_hbm, idx_hbm, out_hbm, idx_v, rows_v, gsem):
    wid = lax.axis_index("s") * NUM_CORES + lax.axis_index("c")
    base = wid * b_per_w
    pltpu.sync_copy(idx_hbm.at[pl.ds(base, b_per_w)], idx_v)
    for c in range(n_chunks):
      pltpu.async_copy(table_hbm.at[idx_v.at[pl.ds(c * CHUNK, CHUNK)]],
                       rows_v, gsem).wait()
      pltpu.sync_copy(rows_v, out_hbm.at[pl.ds(base + c * CHUNK, CHUNK)])

  return gather


def _proj_body(h_ref, w_ref, o_ref):
  o_ref[...] = lax.dot_general(
      h_ref[...], w_ref[...],
      dimension_numbers=(((1,), (1,)), ((), ())),
      preferred_element_type=jnp.float32)


def _project(h, w, block_rows=512):
  B = h.shape[0]
  assert B % block_rows == 0
  return pl.pallas_call(
      _proj_body,
      grid=(B // block_rows,),
      in_specs=[
          pl.BlockSpec((block_rows, R), lambda i: (i, 0)),
          pl.BlockSpec((N_EMBD, R), lambda i: (0, 0)),
      ],
      out_specs=pl.BlockSpec((block_rows, N_EMBD), lambda i: (i, 0)),
      out_shape=jax.ShapeDtypeStruct((B, N_EMBD), jnp.float32),
  )(h, w)


def kernel(x, embed_in_weight, embed_out_weight):
  Bo, L = x.shape
  B = Bo * L
  xf = x.reshape(B).astype(jnp.int32)
  h = _make_gather(B)(embed_in_weight, xf)
  y = _project(h, embed_out_weight)
  return y.reshape(Bo, L, N_EMBD)


# speedup vs baseline: 2.7246x; 1.3230x over previous
import functools

import jax
import jax.numpy as jnp
from jax import lax
from jax.experimental import pallas as pl
from jax.experimental.pallas import tpu as pltpu
from jax.experimental.pallas import tpu_sc as plsc

VOCAB = 1000000
N_EMBD = 1024
R = 128

NUM_CORES = 2
NUM_SUBCORES = 16
NW = NUM_CORES * NUM_SUBCORES

CHUNK = 80


def _make_gather(B):
  assert B % (8 * NW) == 0
  b_per_w = B // NW
  assert b_per_w % CHUNK == 0
  n_chunks = b_per_w // CHUNK
  mesh = plsc.VectorSubcoreMesh(core_axis_name="c", subcore_axis_name="s")

  @functools.partial(
      pl.kernel,
      out_type=jax.ShapeDtypeStruct((B, R), jnp.float32),
      mesh=mesh,
      compiler_params=pltpu.CompilerParams(use_tc_tiling_on_sc=True),
      scratch_types=[
          pltpu.VMEM((b_per_w,), jnp.int32),
          pltpu.VMEM((CHUNK, R), jnp.float32),
          pltpu.SemaphoreType.DMA,
      ],
  )
  def gather(table_hbm, idx_hbm, out_hbm, idx_v, rows_v, gsem):
    wid = lax.axis_index("s") * NUM_CORES + lax.axis_index("c")
    base = wid * b_per_w
    pltpu.sync_copy(idx_hbm.at[pl.ds(base, b_per_w)], idx_v)
    for c in range(n_chunks):
      pltpu.async_copy(table_hbm.at[idx_v.at[pl.ds(c * CHUNK, CHUNK)]],
                       rows_v, gsem).wait()
      pltpu.sync_copy(rows_v, out_hbm.at[pl.ds(base + c * CHUNK, CHUNK)])

  return gather


def _proj_body(L, G, h_ref, w_ref, o_ref):
  y2 = lax.dot_general(
      h_ref[...], w_ref[...],
      dimension_numbers=(((1,), (1,)), ((), ())),
      preferred_element_type=jnp.float32)
  o_ref[...] = y2.reshape(G, L, N_EMBD)


def _project(h, w, Bo, L, seqs_per_block=8):
  G = seqs_per_block
  assert Bo % G == 0 and (G * L) % 8 == 0
  return pl.pallas_call(
      functools.partial(_proj_body, L, G),
      grid=(Bo // G,),
      in_specs=[
          pl.BlockSpec((G * L, R), lambda i: (i, 0)),
          pl.BlockSpec((N_EMBD, R), lambda i: (0, 0)),
      ],
      out_specs=pl.BlockSpec((G, L, N_EMBD), lambda i: (i, 0, 0)),
      out_shape=jax.ShapeDtypeStruct((Bo, L, N_EMBD), jnp.float32),
  )(h, w)


def kernel(x, embed_in_weight, embed_out_weight):
  Bo, L = x.shape
  B = Bo * L
  xf = x.reshape(B).astype(jnp.int32)
  h = _make_gather(B)(embed_in_weight, xf)
  return _project(h, embed_out_weight, Bo, L)


# speedup vs baseline: 3.1998x; 1.1744x over previous
import functools

import jax
import jax.numpy as jnp
from jax import lax
from jax.experimental import pallas as pl
from jax.experimental.pallas import tpu as pltpu
from jax.experimental.pallas import tpu_sc as plsc

VOCAB = 1000000
N_EMBD = 1024
R = 128

NUM_CORES = 2
NUM_SUBCORES = 16
NW = NUM_CORES * NUM_SUBCORES

CHUNK = 80


def _make_gather(B):
  assert B % (8 * NW) == 0
  b_per_w = B // NW
  assert b_per_w % CHUNK == 0
  n_chunks = b_per_w // CHUNK
  mesh = plsc.VectorSubcoreMesh(core_axis_name="c", subcore_axis_name="s")

  @functools.partial(
      pl.kernel,
      out_type=jax.ShapeDtypeStruct((B, R), jnp.float32),
      mesh=mesh,
      compiler_params=pltpu.CompilerParams(use_tc_tiling_on_sc=True),
      scratch_types=[
          pltpu.VMEM((b_per_w,), jnp.int32),
          pltpu.VMEM((CHUNK, R), jnp.float32),
          pltpu.SemaphoreType.DMA,
      ],
  )
  def gather(table_hbm, idx_hbm, out_hbm, idx_v, rows_v, gsem):
    wid = lax.axis_index("s") * NUM_CORES + lax.axis_index("c")
    base = wid * b_per_w
    pltpu.sync_copy(idx_hbm.at[pl.ds(base, b_per_w)], idx_v)
    for c in range(n_chunks):
      pltpu.async_copy(table_hbm.at[idx_v.at[pl.ds(c * CHUNK, CHUNK)]],
                       rows_v, gsem).wait()
      pltpu.sync_copy(rows_v, out_hbm.at[pl.ds(base + c * CHUNK, CHUNK)])

  return gather


def _proj_body(L, G, h_ref, w_ref, o_ref):
  y2 = lax.dot_general(
      h_ref[...], w_ref[...],
      dimension_numbers=(((1,), (1,)), ((), ())),
      preferred_element_type=jnp.float32)
  o_ref[...] = y2.reshape(G, L, N_EMBD)


def _project(h, w, Bo, L, seqs_per_block=8):
  G = seqs_per_block
  assert Bo % G == 0 and (G * L) % 8 == 0
  return pl.pallas_call(
      functools.partial(_proj_body, L, G),
      grid=(Bo // G,),
      in_specs=[
          pl.BlockSpec((G * L, R), lambda i: (i, 0)),
          pl.BlockSpec((N_EMBD, R), lambda i: (0, 0)),
      ],
      out_specs=pl.BlockSpec((G, L, N_EMBD), lambda i: (i, 0, 0)),
      out_shape=jax.ShapeDtypeStruct((Bo, L, N_EMBD), jnp.float32),
  )(h, w)


def kernel(x, embed_in_weight, embed_out_weight):
  Bo, L = x.shape
  B = Bo * L
  xf = x.reshape(B).astype(jnp.int32)
  h = _make_gather(B)(embed_in_weight, xf)
  return _project(h, embed_out_weight, Bo, L, seqs_per_block=32)


# speedup vs baseline: 3.2188x; 1.0060x over previous
import functools

import jax
import jax.numpy as jnp
from jax import lax
from jax.experimental import pallas as pl
from jax.experimental.pallas import tpu as pltpu
from jax.experimental.pallas import tpu_sc as plsc

VOCAB = 1000000
N_EMBD = 1024
R = 128

NUM_CORES = 2
NUM_SUBCORES = 16
NW = NUM_CORES * NUM_SUBCORES

CHUNK = 80


def _make_gather(B):
  assert B % (8 * NW) == 0
  b_per_w = B // NW
  assert b_per_w % CHUNK == 0
  n_chunks = b_per_w // CHUNK
  mesh = plsc.VectorSubcoreMesh(core_axis_name="c", subcore_axis_name="s")

  @functools.partial(
      pl.kernel,
      out_type=jax.ShapeDtypeStruct((B, R), jnp.float32),
      mesh=mesh,
      compiler_params=pltpu.CompilerParams(use_tc_tiling_on_sc=True),
      scratch_types=[
          pltpu.VMEM((b_per_w,), jnp.int32),
          pltpu.VMEM((CHUNK, R), jnp.float32),
          pltpu.SemaphoreType.DMA,
      ],
  )
  def gather(table_hbm, idx_hbm, out_hbm, idx_v, rows_v, gsem):
    wid = lax.axis_index("s") * NUM_CORES + lax.axis_index("c")
    base = wid * b_per_w
    pltpu.sync_copy(idx_hbm.at[pl.ds(base, b_per_w)], idx_v)
    for c in range(n_chunks):
      pltpu.async_copy(table_hbm.at[idx_v.at[pl.ds(c * CHUNK, CHUNK)]],
                       rows_v, gsem).wait()
      pltpu.sync_copy(rows_v, out_hbm.at[pl.ds(base + c * CHUNK, CHUNK)])

  return gather


def _proj_body(L, G, h_ref, w_ref, o_ref):
  y2 = lax.dot_general(
      h_ref[...], w_ref[...],
      dimension_numbers=(((1,), (1,)), ((), ())),
      preferred_element_type=jnp.float32)
  o_ref[...] = y2.reshape(G, L, N_EMBD)


def _project(h, w, Bo, L, seqs_per_block=8):
  G = seqs_per_block
  assert Bo % G == 0 and (G * L) % 8 == 0
  return pl.pallas_call(
      functools.partial(_proj_body, L, G),
      grid=(Bo // G,),
      in_specs=[
          pl.BlockSpec((G * L, R), lambda i: (i, 0)),
          pl.BlockSpec((N_EMBD, R), lambda i: (0, 0)),
      ],
      out_specs=pl.BlockSpec((G, L, N_EMBD), lambda i: (i, 0, 0)),
      out_shape=jax.ShapeDtypeStruct((Bo, L, N_EMBD), jnp.float32),
  )(h, w)


def kernel(x, embed_in_weight, embed_out_weight):
  Bo, L = x.shape
  B = Bo * L
  xf = x.reshape(B).astype(jnp.int32)
  h = _make_gather(B)(embed_in_weight, xf)
  return _project(h, embed_out_weight, Bo, L, seqs_per_block=64)
